# baseline (speedup 1.0000x reference)
import jax
import jax.numpy as jnp
from jax import lax
from jax.experimental import pallas as pl
from jax.experimental.pallas import tpu as pltpu
from jax.experimental.pallas import tpu_sc as plsc

S, D, H, E, K = 2048, 768, 1536, 8, 2
TB = 256
NT = S // TB
BLK = 256
A = S * K
TP = A + E * BLK
NB = TP // BLK
NBP = 32

NW = 32
TPW = S // NW

NEG_INF = float("-inf")


CH = 512
NCH = A // CH


def _gating_kernel(x_ref, wg_ref, bg_ref,
                   gs_ref, ts_ref, ti_ref, w0_ref, w1_ref,
                   p0_ref, p1_ref, be_ref):
    eidx = jax.lax.broadcasted_iota(jnp.int32, (S, E), 1)
    gs = jnp.dot(x_ref[...], wg_ref[...], preferred_element_type=jnp.float32)
    gs = gs + bg_ref[...]
    gs_ref[...] = gs
    m = jnp.max(gs, axis=-1, keepdims=True)
    ex = jnp.exp(gs - m)
    soft = ex / jnp.sum(ex, axis=-1, keepdims=True)
    m1 = jnp.max(soft, axis=-1, keepdims=True)
    i1 = jnp.min(jnp.where(soft == m1, eidx, E), axis=-1, keepdims=True)
    soft2 = jnp.where(eidx == i1, NEG_INF, soft)
    m2 = jnp.max(soft2, axis=-1, keepdims=True)
    i2 = jnp.min(jnp.where(soft2 == m2, eidx, E), axis=-1, keepdims=True)
    ts_ref[...] = jnp.concatenate([m1, m2], axis=-1)
    ti_ref[...] = jnp.concatenate([i1, i2], axis=-1)
    ee = jnp.exp(m2 - m1)
    w0_ref[...] = jnp.broadcast_to(1.0 / (1.0 + ee), (S, 16))
    w1_ref[...] = jnp.broadcast_to(ee / (1.0 + ee), (S, 16))

    oh0 = (eidx == i1).astype(jnp.float32)
    oh1 = (eidx == i2).astype(jnp.float32)
    stacked = jnp.concatenate([oh0, oh1], axis=0)
    ii = jax.lax.broadcasted_iota(jnp.int32, (CH, CH), 0)
    jj = jax.lax.broadcasted_iota(jnp.int32, (CH, CH), 1)
    tri = (jj < ii).astype(jnp.float32)
    parts = []
    run = jnp.zeros((1, E), jnp.float32)
    for c in range(NCH):
        blk = stacked[c * CH:(c + 1) * CH]
        parts.append(jnp.dot(tri, blk,
                             preferred_element_type=jnp.float32) + run)
        run = run + jnp.sum(blk, axis=0, keepdims=True)
    excl = jnp.concatenate(parts, axis=0)
    r0 = jnp.sum(excl[:S] * oh0, axis=-1)
    r1 = jnp.sum(excl[S:] * oh1, axis=-1)

    cnt_i = run.astype(jnp.int32)
    padded = jnp.bitwise_and(cnt_i + (BLK - 1), ~(BLK - 1))
    ei = jax.lax.broadcasted_iota(jnp.int32, (E, E), 0)
    ej = jax.lax.broadcasted_iota(jnp.int32, (E, E), 1)
    tri_le = (ei <= ej).astype(jnp.float32)
    incl = jnp.dot(padded.astype(jnp.float32), tri_le,
                   preferred_element_type=jnp.float32).astype(jnp.int32)
    offs = (incl - padded).astype(jnp.float32)
    off0 = jnp.sum(oh0 * offs, axis=-1)
    off1 = jnp.sum(oh1 * offs, axis=-1)
    p0_ref[...] = (off0 + r0).astype(jnp.int32)[:, None]
    p1_ref[...] = (off1 + r1).astype(jnp.int32)[:, None]
    ib = jax.lax.broadcasted_iota(jnp.int32, (NBP, E), 0) * BLK
    le = jax.lax.broadcasted_iota(jnp.int32, (NBP, E), 1)
    hits = jnp.where((ib >= incl) & (le < E - 1), 1, 0)
    be_ref[...] = jnp.sum(hits, axis=-1).astype(jnp.int32)[None, :]


def _gating(x2, Wg, bg):
    return pl.pallas_call(
        _gating_kernel,
        out_shape=[
            jax.ShapeDtypeStruct((S, E), jnp.float32),
            jax.ShapeDtypeStruct((S, K), jnp.float32),
            jax.ShapeDtypeStruct((S, K), jnp.int32),
            jax.ShapeDtypeStruct((S, 16), jnp.float32),
            jax.ShapeDtypeStruct((S, 16), jnp.float32),
            jax.ShapeDtypeStruct((S, 1), jnp.int32),
            jax.ShapeDtypeStruct((S, 1), jnp.int32),
            jax.ShapeDtypeStruct((1, NBP), jnp.int32),
        ],
    )(x2, Wg, bg.reshape(1, E))


def _route_body(xh, p0h, p1h, gxh, p0_v, p1_v, xbuf, sem0, sem1):
    wid = lax.axis_index("s") * 2 + lax.axis_index("c")
    base = wid * TPW
    pltpu.sync_copy(p0h.at[pl.ds(base, TPW)], p0_v)
    pltpu.sync_copy(p1h.at[pl.ds(base, TPW)], p1_v)
    pltpu.sync_copy(xh.at[pl.ds(base, TPW)], xbuf)
    cp0 = pltpu.async_copy(xbuf, gxh.at[p0_v], sem0)
    cp1 = pltpu.async_copy(xbuf, gxh.at[p1_v], sem1)
    cp0.wait()
    cp1.wait()


def _route(x2, p0, p1):
    f = pl.kernel(
        _route_body,
        out_type=jax.ShapeDtypeStruct((TP, D), jnp.float32),
        mesh=plsc.VectorSubcoreMesh(core_axis_name="c", subcore_axis_name="s"),
        scratch_types=[
            pltpu.VMEM((TPW,), jnp.int32),
            pltpu.VMEM((TPW,), jnp.int32),
            pltpu.VMEM((TPW, D), jnp.float32),
            pltpu.SemaphoreType.DMA,
            pltpu.SemaphoreType.DMA,
        ],
    )
    return f(x2, p0, p1)


def _erf(z):
    a1, a2, a3 = 0.254829592, -0.284496736, 1.421413741
    a4, a5, p = -1.453152027, 1.061405429, 0.3275911
    s = jnp.sign(z)
    za = jnp.abs(z)
    t = 1.0 / (1.0 + p * za)
    poly = ((((a5 * t + a4) * t + a3) * t + a2) * t + a1) * t
    return s * (1.0 - poly * jnp.exp(-za * za))


def _gelu_exact(v):
    return 0.5 * v * (1.0 + _erf(v * 0.7071067811865476))


def _mlp_kernel(be_ref, xg_ref, w1_ref, b1_ref, w2_ref, b2_ref, out_ref,
                w1b_s, w2b_s):
    i = pl.program_id(0)

    @pl.when((i == 0) | (be_ref[i] != be_ref[jnp.maximum(i - 1, 0)]))
    def _():
        w1b_s[...] = w1_ref[0].astype(jnp.bfloat16)
        w2b_s[...] = w2_ref[0].astype(jnp.bfloat16)

    xb = xg_ref[...].astype(jnp.bfloat16)
    h = jnp.dot(xb, w1b_s[...], preferred_element_type=jnp.float32)
    h = _gelu_exact(h + b1_ref[0])
    o = jnp.dot(h.astype(jnp.bfloat16), w2b_s[...],
                preferred_element_type=jnp.float32)
    out_ref[...] = o + b2_ref[0]


def _mlp(be, xg, W1, b1, W2, b2):
    grid_spec = pltpu.PrefetchScalarGridSpec(
        num_scalar_prefetch=1,
        grid=(NB,),
        in_specs=[
            pl.BlockSpec((BLK, D), lambda i, be: (i, 0)),
            pl.BlockSpec((1, D, H), lambda i, be: (be[i], 0, 0)),
            pl.BlockSpec((1, 1, H), lambda i, be: (be[i], 0, 0)),
            pl.BlockSpec((1, H, D), lambda i, be: (be[i], 0, 0)),
            pl.BlockSpec((1, 1, D), lambda i, be: (be[i], 0, 0)),
        ],
        out_specs=pl.BlockSpec((BLK, D), lambda i, be: (i, 0)),
        scratch_shapes=[
            pltpu.VMEM((D, H), jnp.bfloat16),
            pltpu.VMEM((H, D), jnp.bfloat16),
        ],
    )
    return pl.pallas_call(
        _mlp_kernel,
        grid_spec=grid_spec,
        out_shape=jax.ShapeDtypeStruct((TP, D), jnp.float32),
    )(be, xg, W1, b1.reshape(E, 1, H), W2, b2.reshape(E, 1, D))


def _combine_body(rowsh, p0h, p1h, w0h, w1h, yh,
                  p0_v, p1_v, w0_v, w1_v, buf0, buf1, sem0, sem1):
    wid = lax.axis_index("s") * 2 + lax.axis_index("c")
    base = wid * TPW
    pltpu.sync_copy(p0h.at[pl.ds(base, TPW)], p0_v)
    pltpu.sync_copy(p1h.at[pl.ds(base, TPW)], p1_v)
    pltpu.sync_copy(w0h.at[pl.ds(base, TPW)], w0_v)
    pltpu.sync_copy(w1h.at[pl.ds(base, TPW)], w1_v)
    cp0 = pltpu.async_copy(rowsh.at[p0_v], buf0, sem0)
    cp1 = pltpu.async_copy(rowsh.at[p1_v], buf1, sem1)
    cp0.wait()
    cp1.wait()

    def body(j, carry):
        a = w0_v[j]
        b = w1_v[j]
        for k in range(D // 16):
            sl = pl.ds(k * 16, 16)
            buf0[j, sl] = a * buf0[j, sl] + b * buf1[j, sl]
        return carry

    lax.fori_loop(0, TPW, body, 0)
    pltpu.sync_copy(buf0, yh.at[pl.ds(base, TPW)])


def _combine(rows, p0, p1, w0, w1):
    f = pl.kernel(
        _combine_body,
        out_type=jax.ShapeDtypeStruct((S, D), jnp.float32),
        mesh=plsc.VectorSubcoreMesh(core_axis_name="c", subcore_axis_name="s"),
        scratch_types=[
            pltpu.VMEM((TPW,), jnp.int32),
            pltpu.VMEM((TPW,), jnp.int32),
            pltpu.VMEM((TPW, 16), jnp.float32),
            pltpu.VMEM((TPW, 16), jnp.float32),
            pltpu.VMEM((TPW, D), jnp.float32),
            pltpu.VMEM((TPW, D), jnp.float32),
            pltpu.SemaphoreType.DMA,
            pltpu.SemaphoreType.DMA,
        ],
    )
    return f(rows, p0, p1, w0, w1)


def kernel(x, W1, b1, W2, b2, Wg, bg):
    x2 = x.reshape(S, D)
    (gs, ts, ti, w0, w1, p0, p1, be) = _gating(x2, Wg, bg)
    gx = _route(x2, p0.reshape(S), p1.reshape(S))
    rows = _mlp(be.reshape(NBP), gx, W1, b1, W2, b2)
    y = _combine(rows, p0.reshape(S), p1.reshape(S), w0, w1)
    return (y.reshape(1, S, D),
            (ts.reshape(1, S, K), ti.reshape(1, S, K), gs.reshape(1, S, E)))


# speedup vs baseline: 1.0819x; 1.0819x over previous
import jax
import jax.numpy as jnp
from jax import lax
from jax.experimental import pallas as pl
from jax.experimental.pallas import tpu as pltpu
from jax.experimental.pallas import tpu_sc as plsc

S, D, H, E, K = 2048, 768, 1536, 8, 2
TB = 256
NT = S // TB
BLK = 256
A = S * K
TP = A + E * BLK
NB = TP // BLK
NBP = 32

NW = 32
TPW = S // NW

NEG_INF = float("-inf")


CH = 512
NCH = A // CH


def _gating_kernel(x_ref, wg_ref, bg_ref,
                   gs_ref, ts_ref, ti_ref, w0_ref, w1_ref,
                   p0_ref, p1_ref, be_ref):
    eidx = jax.lax.broadcasted_iota(jnp.int32, (S, E), 1)
    gs = jnp.dot(x_ref[...], wg_ref[...], preferred_element_type=jnp.float32)
    gs = gs + bg_ref[...]
    gs_ref[...] = gs
    m = jnp.max(gs, axis=-1, keepdims=True)
    ex = jnp.exp(gs - m)
    soft = ex / jnp.sum(ex, axis=-1, keepdims=True)
    m1 = jnp.max(soft, axis=-1, keepdims=True)
    i1 = jnp.min(jnp.where(soft == m1, eidx, E), axis=-1, keepdims=True)
    soft2 = jnp.where(eidx == i1, NEG_INF, soft)
    m2 = jnp.max(soft2, axis=-1, keepdims=True)
    i2 = jnp.min(jnp.where(soft2 == m2, eidx, E), axis=-1, keepdims=True)
    ts_ref[...] = jnp.concatenate([m1, m2], axis=-1)
    ti_ref[...] = jnp.concatenate([i1, i2], axis=-1)
    ee = jnp.exp(m2 - m1)
    w0_ref[...] = jnp.broadcast_to(1.0 / (1.0 + ee), (S, 16))
    w1_ref[...] = jnp.broadcast_to(ee / (1.0 + ee), (S, 16))

    oh0 = (eidx == i1).astype(jnp.float32)
    oh1 = (eidx == i2).astype(jnp.float32)
    stacked = jnp.concatenate([oh0, oh1], axis=0)
    ii = jax.lax.broadcasted_iota(jnp.int32, (CH, CH), 0)
    jj = jax.lax.broadcasted_iota(jnp.int32, (CH, CH), 1)
    tri = (jj < ii).astype(jnp.float32)
    parts = []
    run = jnp.zeros((1, E), jnp.float32)
    for c in range(NCH):
        blk = stacked[c * CH:(c + 1) * CH]
        parts.append(jnp.dot(tri, blk,
                             preferred_element_type=jnp.float32) + run)
        run = run + jnp.sum(blk, axis=0, keepdims=True)
    excl = jnp.concatenate(parts, axis=0)
    r0 = jnp.sum(excl[:S] * oh0, axis=-1)
    r1 = jnp.sum(excl[S:] * oh1, axis=-1)

    cnt_i = run.astype(jnp.int32)
    padded = jnp.bitwise_and(cnt_i + (BLK - 1), ~(BLK - 1))
    ei = jax.lax.broadcasted_iota(jnp.int32, (E, E), 0)
    ej = jax.lax.broadcasted_iota(jnp.int32, (E, E), 1)
    tri_le = (ei <= ej).astype(jnp.float32)
    incl = jnp.dot(padded.astype(jnp.float32), tri_le,
                   preferred_element_type=jnp.float32).astype(jnp.int32)
    offs = (incl - padded).astype(jnp.float32)
    off0 = jnp.sum(oh0 * offs, axis=-1)
    off1 = jnp.sum(oh1 * offs, axis=-1)
    p0_ref[...] = (off0 + r0).astype(jnp.int32)[:, None]
    p1_ref[...] = (off1 + r1).astype(jnp.int32)[:, None]
    ib = jax.lax.broadcasted_iota(jnp.int32, (NBP, E), 0) * BLK
    le = jax.lax.broadcasted_iota(jnp.int32, (NBP, E), 1)
    hits = jnp.where((ib >= incl) & (le < E - 1), 1, 0)
    bes = jnp.sum(hits, axis=-1).astype(jnp.int32)[None, :]
    lane_b = jax.lax.broadcasted_iota(jnp.int32, (1, NBP), 1)
    total = jnp.sum(padded, axis=-1, keepdims=True)
    be_ref[...] = jnp.where(lane_b == NB, total, bes)


def _gating(x2, Wg, bg):
    return pl.pallas_call(
        _gating_kernel,
        out_shape=[
            jax.ShapeDtypeStruct((S, E), jnp.float32),
            jax.ShapeDtypeStruct((S, K), jnp.float32),
            jax.ShapeDtypeStruct((S, K), jnp.int32),
            jax.ShapeDtypeStruct((S, 16), jnp.float32),
            jax.ShapeDtypeStruct((S, 16), jnp.float32),
            jax.ShapeDtypeStruct((S, 1), jnp.int32),
            jax.ShapeDtypeStruct((S, 1), jnp.int32),
            jax.ShapeDtypeStruct((1, NBP), jnp.int32),
        ],
    )(x2, Wg, bg.reshape(1, E))


def _route_body(xh, p0h, p1h, gxh, p0_v, p1_v, xbuf, sem0, sem1):
    wid = lax.axis_index("s") * 2 + lax.axis_index("c")
    base = wid * TPW
    pltpu.sync_copy(p0h.at[pl.ds(base, TPW)], p0_v)
    pltpu.sync_copy(p1h.at[pl.ds(base, TPW)], p1_v)
    pltpu.sync_copy(xh.at[pl.ds(base, TPW)], xbuf)
    cp0 = pltpu.async_copy(xbuf, gxh.at[p0_v], sem0)
    cp1 = pltpu.async_copy(xbuf, gxh.at[p1_v], sem1)
    cp0.wait()
    cp1.wait()


def _route(x2, p0, p1):
    f = pl.kernel(
        _route_body,
        out_type=jax.ShapeDtypeStruct((TP, D), jnp.float32),
        mesh=plsc.VectorSubcoreMesh(core_axis_name="c", subcore_axis_name="s"),
        scratch_types=[
            pltpu.VMEM((TPW,), jnp.int32),
            pltpu.VMEM((TPW,), jnp.int32),
            pltpu.VMEM((TPW, D), jnp.float32),
            pltpu.SemaphoreType.DMA,
            pltpu.SemaphoreType.DMA,
        ],
    )
    return f(x2, p0, p1)


def _erf(z):
    a1, a2, a3 = 0.254829592, -0.284496736, 1.421413741
    a4, a5, p = -1.453152027, 1.061405429, 0.3275911
    s = jnp.sign(z)
    za = jnp.abs(z)
    t = 1.0 / (1.0 + p * za)
    poly = ((((a5 * t + a4) * t + a3) * t + a2) * t + a1) * t
    return s * (1.0 - poly * jnp.exp(-za * za))


def _gelu_exact(v):
    return 0.5 * v * (1.0 + _erf(v * 0.7071067811865476))


def _mlp_kernel(be_ref, xg_ref, w1_ref, b1_ref, w2_ref, b2_ref, out_ref):
    i = pl.program_id(0)

    @pl.when(i * BLK < be_ref[NB])
    def _():
        h = jnp.dot(xg_ref[...], w1_ref[0],
                    preferred_element_type=jnp.float32)
        h = _gelu_exact(h + b1_ref[0])
        o = jnp.dot(h, w2_ref[0], preferred_element_type=jnp.float32)
        out_ref[...] = o + b2_ref[0]


def _mlp(be, xg, W1, b1, W2, b2):
    grid_spec = pltpu.PrefetchScalarGridSpec(
        num_scalar_prefetch=1,
        grid=(NB,),
        in_specs=[
            pl.BlockSpec((BLK, D), lambda i, be: (i, 0)),
            pl.BlockSpec((1, D, H), lambda i, be: (be[i], 0, 0)),
            pl.BlockSpec((1, 1, H), lambda i, be: (be[i], 0, 0)),
            pl.BlockSpec((1, H, D), lambda i, be: (be[i], 0, 0)),
            pl.BlockSpec((1, 1, D), lambda i, be: (be[i], 0, 0)),
        ],
        out_specs=pl.BlockSpec((BLK, D), lambda i, be: (i, 0)),
    )
    return pl.pallas_call(
        _mlp_kernel,
        grid_spec=grid_spec,
        out_shape=jax.ShapeDtypeStruct((TP, D), jnp.float32),
    )(be, xg, W1, b1.reshape(E, 1, H), W2, b2.reshape(E, 1, D))


def _combine_body(rowsh, p0h, p1h, w0h, w1h, yh,
                  p0_v, p1_v, w0_v, w1_v, buf0, buf1, sem0, sem1):
    wid = lax.axis_index("s") * 2 + lax.axis_index("c")
    base = wid * TPW
    pltpu.sync_copy(p0h.at[pl.ds(base, TPW)], p0_v)
    pltpu.sync_copy(p1h.at[pl.ds(base, TPW)], p1_v)
    pltpu.sync_copy(w0h.at[pl.ds(base, TPW)], w0_v)
    pltpu.sync_copy(w1h.at[pl.ds(base, TPW)], w1_v)
    cp0 = pltpu.async_copy(rowsh.at[p0_v], buf0, sem0)
    cp1 = pltpu.async_copy(rowsh.at[p1_v], buf1, sem1)
    cp0.wait()
    cp1.wait()

    def body(j, carry):
        a = w0_v[j]
        b = w1_v[j]
        for k in range(D // 16):
            sl = pl.ds(k * 16, 16)
            buf0[j, sl] = a * buf0[j, sl] + b * buf1[j, sl]
        return carry

    lax.fori_loop(0, TPW, body, 0)
    pltpu.sync_copy(buf0, yh.at[pl.ds(base, TPW)])


def _combine(rows, p0, p1, w0, w1):
    f = pl.kernel(
        _combine_body,
        out_type=jax.ShapeDtypeStruct((S, D), jnp.float32),
        mesh=plsc.VectorSubcoreMesh(core_axis_name="c", subcore_axis_name="s"),
        scratch_types=[
            pltpu.VMEM((TPW,), jnp.int32),
            pltpu.VMEM((TPW,), jnp.int32),
            pltpu.VMEM((TPW, 16), jnp.float32),
            pltpu.VMEM((TPW, 16), jnp.float32),
            pltpu.VMEM((TPW, D), jnp.float32),
            pltpu.VMEM((TPW, D), jnp.float32),
            pltpu.SemaphoreType.DMA,
            pltpu.SemaphoreType.DMA,
        ],
    )
    return f(rows, p0, p1, w0, w1)


def kernel(x, W1, b1, W2, b2, Wg, bg):
    x2 = x.reshape(S, D)
    (gs, ts, ti, w0, w1, p0, p1, be) = _gating(x2, Wg, bg)
    gx = _route(x2, p0.reshape(S), p1.reshape(S))
    rows = _mlp(be.reshape(NBP), gx, W1, b1, W2, b2)
    y = _combine(rows, p0.reshape(S), p1.reshape(S), w0, w1)
    return (y.reshape(1, S, D),
            (ts.reshape(1, S, K), ti.reshape(1, S, K), gs.reshape(1, S, E)))


# speedup vs baseline: 1.2217x; 1.1292x over previous
import jax
import jax.numpy as jnp
from jax import lax
from jax.experimental import pallas as pl
from jax.experimental.pallas import tpu as pltpu
from jax.experimental.pallas import tpu_sc as plsc

S, D, H, E, K = 2048, 768, 1536, 8, 2
TB = 256
NT = S // TB
BLK = 256
A = S * K
TP = A + E * BLK
NB = TP // BLK
NBP = 32

NW = 32
TPW = S // NW

NEG_INF = float("-inf")


CH = 512
NCH = A // CH


def _gating_kernel(x_ref, wg_ref, bg_ref,
                   gs_ref, ts_ref, ti_ref, w0_ref, w1_ref,
                   p0_ref, p1_ref, be_ref):
    eidx = jax.lax.broadcasted_iota(jnp.int32, (S, E), 1)
    gs = jnp.dot(x_ref[...], wg_ref[...], preferred_element_type=jnp.float32)
    gs = gs + bg_ref[...]
    gs_ref[...] = gs
    m = jnp.max(gs, axis=-1, keepdims=True)
    ex = jnp.exp(gs - m)
    soft = ex / jnp.sum(ex, axis=-1, keepdims=True)
    m1 = jnp.max(soft, axis=-1, keepdims=True)
    i1 = jnp.min(jnp.where(soft == m1, eidx, E), axis=-1, keepdims=True)
    soft2 = jnp.where(eidx == i1, NEG_INF, soft)
    m2 = jnp.max(soft2, axis=-1, keepdims=True)
    i2 = jnp.min(jnp.where(soft2 == m2, eidx, E), axis=-1, keepdims=True)
    ts_ref[...] = jnp.concatenate([m1, m2], axis=-1)
    ti_ref[...] = jnp.concatenate([i1, i2], axis=-1)
    ee = jnp.exp(m2 - m1)
    w0_ref[...] = jnp.broadcast_to(1.0 / (1.0 + ee), (S, 16))
    w1_ref[...] = jnp.broadcast_to(ee / (1.0 + ee), (S, 16))

    oh0 = (eidx == i1).astype(jnp.float32)
    oh1 = (eidx == i2).astype(jnp.float32)
    stacked = jnp.concatenate([oh0, oh1], axis=0)
    ii = jax.lax.broadcasted_iota(jnp.int32, (CH, CH), 0)
    jj = jax.lax.broadcasted_iota(jnp.int32, (CH, CH), 1)
    tri = (jj < ii).astype(jnp.float32)
    parts = []
    run = jnp.zeros((1, E), jnp.float32)
    for c in range(NCH):
        blk = stacked[c * CH:(c + 1) * CH]
        parts.append(jnp.dot(tri, blk,
                             preferred_element_type=jnp.float32) + run)
        run = run + jnp.sum(blk, axis=0, keepdims=True)
    excl = jnp.concatenate(parts, axis=0)
    r0 = jnp.sum(excl[:S] * oh0, axis=-1)
    r1 = jnp.sum(excl[S:] * oh1, axis=-1)

    cnt_i = run.astype(jnp.int32)
    padded = jnp.bitwise_and(cnt_i + (BLK - 1), ~(BLK - 1))
    ei = jax.lax.broadcasted_iota(jnp.int32, (E, E), 0)
    ej = jax.lax.broadcasted_iota(jnp.int32, (E, E), 1)
    tri_le = (ei <= ej).astype(jnp.float32)
    incl = jnp.dot(padded.astype(jnp.float32), tri_le,
                   preferred_element_type=jnp.float32).astype(jnp.int32)
    offs = (incl - padded).astype(jnp.float32)
    off0 = jnp.sum(oh0 * offs, axis=-1)
    off1 = jnp.sum(oh1 * offs, axis=-1)
    p0_ref[...] = (off0 + r0).astype(jnp.int32)[:, None]
    p1_ref[...] = (off1 + r1).astype(jnp.int32)[:, None]
    ib = jax.lax.broadcasted_iota(jnp.int32, (NBP, E), 0) * BLK
    le = jax.lax.broadcasted_iota(jnp.int32, (NBP, E), 1)
    hits = jnp.where((ib >= incl) & (le < E - 1), 1, 0)
    bes = jnp.sum(hits, axis=-1).astype(jnp.int32)[None, :]
    lane_b = jax.lax.broadcasted_iota(jnp.int32, (1, NBP), 1)
    total = jnp.sum(padded, axis=-1, keepdims=True)
    be_ref[...] = jnp.where(lane_b == NB, total, bes)


def _gating(x2, Wg, bg):
    return pl.pallas_call(
        _gating_kernel,
        out_shape=[
            jax.ShapeDtypeStruct((S, E), jnp.float32),
            jax.ShapeDtypeStruct((S, K), jnp.float32),
            jax.ShapeDtypeStruct((S, K), jnp.int32),
            jax.ShapeDtypeStruct((S, 16), jnp.float32),
            jax.ShapeDtypeStruct((S, 16), jnp.float32),
            jax.ShapeDtypeStruct((S, 1), jnp.int32),
            jax.ShapeDtypeStruct((S, 1), jnp.int32),
            jax.ShapeDtypeStruct((1, NBP), jnp.int32),
        ],
    )(x2, Wg, bg.reshape(1, E))


def _route_body(xh, p0h, p1h, gxh, p0_v, p1_v, xbuf, sem0, sem1):
    wid = lax.axis_index("s") * 2 + lax.axis_index("c")
    base = wid * TPW
    pltpu.sync_copy(p0h.at[pl.ds(base, TPW)], p0_v)
    pltpu.sync_copy(p1h.at[pl.ds(base, TPW)], p1_v)
    pltpu.sync_copy(xh.at[pl.ds(base, TPW)], xbuf)
    cp0 = pltpu.async_copy(xbuf, gxh.at[p0_v], sem0)
    cp1 = pltpu.async_copy(xbuf, gxh.at[p1_v], sem1)
    cp0.wait()
    cp1.wait()


def _route(x2, p0, p1):
    f = pl.kernel(
        _route_body,
        out_type=jax.ShapeDtypeStruct((TP, D), jnp.float32),
        mesh=plsc.VectorSubcoreMesh(core_axis_name="c", subcore_axis_name="s"),
        scratch_types=[
            pltpu.VMEM((TPW,), jnp.int32),
            pltpu.VMEM((TPW,), jnp.int32),
            pltpu.VMEM((TPW, D), jnp.float32),
            pltpu.SemaphoreType.DMA,
            pltpu.SemaphoreType.DMA,
        ],
    )
    return f(x2, p0, p1)


def _erf(z):
    a1, a2, a3 = 0.254829592, -0.284496736, 1.421413741
    a4, a5, p = -1.453152027, 1.061405429, 0.3275911
    s = jnp.sign(z)
    za = jnp.abs(z)
    t = 1.0 / (1.0 + p * za)
    poly = ((((a5 * t + a4) * t + a3) * t + a2) * t + a1) * t
    return s * (1.0 - poly * jnp.exp(-za * za))


def _gelu_exact(v):
    return 0.5 * v * (1.0 + lax.erf(v * 0.7071067811865476))


def _mlp_kernel(be_ref, xg_ref, w1_ref, b1_ref, w2_ref, b2_ref, out_ref):
    i = pl.program_id(0)

    @pl.when(i * BLK < be_ref[NB])
    def _():
        h = jnp.dot(xg_ref[...], w1_ref[0],
                    preferred_element_type=jnp.float32)
        h = _gelu_exact(h + b1_ref[0])
        o = jnp.dot(h, w2_ref[0], preferred_element_type=jnp.float32)
        out_ref[...] = o + b2_ref[0]


def _mlp(be, xg, W1, b1, W2, b2):
    grid_spec = pltpu.PrefetchScalarGridSpec(
        num_scalar_prefetch=1,
        grid=(NB,),
        in_specs=[
            pl.BlockSpec((BLK, D), lambda i, be: (i, 0)),
            pl.BlockSpec((1, D, H), lambda i, be: (be[i], 0, 0)),
            pl.BlockSpec((1, 1, H), lambda i, be: (be[i], 0, 0)),
            pl.BlockSpec((1, H, D), lambda i, be: (be[i], 0, 0)),
            pl.BlockSpec((1, 1, D), lambda i, be: (be[i], 0, 0)),
        ],
        out_specs=pl.BlockSpec((BLK, D), lambda i, be: (i, 0)),
    )
    return pl.pallas_call(
        _mlp_kernel,
        grid_spec=grid_spec,
        out_shape=jax.ShapeDtypeStruct((TP, D), jnp.float32),
    )(be, xg, W1, b1.reshape(E, 1, H), W2, b2.reshape(E, 1, D))


def _combine_body(rowsh, p0h, p1h, w0h, w1h, yh,
                  p0_v, p1_v, w0_v, w1_v, buf0, buf1, sem0, sem1):
    wid = lax.axis_index("s") * 2 + lax.axis_index("c")
    base = wid * TPW
    pltpu.sync_copy(p0h.at[pl.ds(base, TPW)], p0_v)
    pltpu.sync_copy(p1h.at[pl.ds(base, TPW)], p1_v)
    pltpu.sync_copy(w0h.at[pl.ds(base, TPW)], w0_v)
    pltpu.sync_copy(w1h.at[pl.ds(base, TPW)], w1_v)
    cp0 = pltpu.async_copy(rowsh.at[p0_v], buf0, sem0)
    cp1 = pltpu.async_copy(rowsh.at[p1_v], buf1, sem1)
    cp0.wait()
    cp1.wait()

    def body(j, carry):
        a = w0_v[j]
        b = w1_v[j]
        for k in range(D // 16):
            sl = pl.ds(k * 16, 16)
            buf0[j, sl] = a * buf0[j, sl] + b * buf1[j, sl]
        return carry

    lax.fori_loop(0, TPW, body, 0)
    pltpu.sync_copy(buf0, yh.at[pl.ds(base, TPW)])


def _combine(rows, p0, p1, w0, w1):
    f = pl.kernel(
        _combine_body,
        out_type=jax.ShapeDtypeStruct((S, D), jnp.float32),
        mesh=plsc.VectorSubcoreMesh(core_axis_name="c", subcore_axis_name="s"),
        scratch_types=[
            pltpu.VMEM((TPW,), jnp.int32),
            pltpu.VMEM((TPW,), jnp.int32),
            pltpu.VMEM((TPW, 16), jnp.float32),
            pltpu.VMEM((TPW, 16), jnp.float32),
            pltpu.VMEM((TPW, D), jnp.float32),
            pltpu.VMEM((TPW, D), jnp.float32),
            pltpu.SemaphoreType.DMA,
            pltpu.SemaphoreType.DMA,
        ],
    )
    return f(rows, p0, p1, w0, w1)


def kernel(x, W1, b1, W2, b2, Wg, bg):
    x2 = x.reshape(S, D)
    (gs, ts, ti, w0, w1, p0, p1, be) = _gating(x2, Wg, bg)
    gx = _route(x2, p0.reshape(S), p1.reshape(S))
    rows = _mlp(be.reshape(NBP), gx, W1, b1, W2, b2)
    y = _combine(rows, p0.reshape(S), p1.reshape(S), w0, w1)
    return (y.reshape(1, S, D),
            (ts.reshape(1, S, K), ti.reshape(1, S, K), gs.reshape(1, S, E)))
